# Initial kernel scaffold; baseline (speedup 1.0000x reference)
import jax
import jax.numpy as jnp
from jax.experimental import pallas as pl


def kernel(xr, edge_index, edge_attr, W1, b1, a1, W2, b2, a2, W3, b3, a3, W4, b4, root, bias, a_out):
    raise NotImplementedError("write your pallas kernel here")


# speedup vs baseline: 2.3840x; 2.3840x over previous
import functools

import jax
import jax.numpy as jnp
from jax import lax
from jax.experimental import pallas as pl
from jax.experimental.pallas import tpu as pltpu
from jax.experimental.pallas import tpu_sc as plsc

N = 10000
E = 320000
C = 16
KW = 64
DE = 16

NC = 2
NS = 16
NW = NC * NS
EW = E // NW
SEG = 80
CHUNK = 400
NCHUNK = EW // CHUNK
SPC = CHUNK // SEG
ST = 17
NPASS = 2
NHALF = N // NPASS
ACC_W = NHALF * ST


def _gather_body(xrp_hbm, src_hbm, xj_hbm, idx_v, rowsp_v, out_v, sem):
    wid = lax.axis_index("s") * NC + lax.axis_index("c")

    def seg(k, carry):
        base = wid * EW + k * SEG
        pltpu.sync_copy(src_hbm.at[pl.ds(base, SEG)], idx_v)
        pltpu.async_copy(xrp_hbm.at[idx_v], rowsp_v, sem).wait()
        kk = (k % SPC) * SEG

        def extract(r, carry2):
            out_v[kk + r] = rowsp_v[r, pl.ds(0, C)]
            return carry2

        lax.fori_loop(0, SEG, extract, 0)

        @pl.when(k % SPC == SPC - 1)
        def _():
            pltpu.sync_copy(
                out_v, xj_hbm.at[pl.ds(wid * EW + (k - (SPC - 1)) * SEG, CHUNK)]
            )

        return carry

    lax.fori_loop(0, EW // SEG, seg, 0)


def _scatter_body(msgf_hbm, dst_hbm, parts_hbm, idx_v, data_v, acc_v):
    wid = lax.axis_index("s") * NC + lax.axis_index("c")
    iota = lax.iota(jnp.int32, 16)
    mask0 = iota == 0
    ones16 = jnp.ones((16,), jnp.float32)

    def one_pass(p, carry):
        lo = p * NHALF

        def zrow(i, carry2):
            acc_v[pl.ds(i * 16, 16)] = jnp.zeros((16,), jnp.float32)
            return carry2

        lax.fori_loop(0, ACC_W // 16, zrow, 0)

        def chunk(c, carry2):
            base = wid * EW + c * CHUNK
            pltpu.sync_copy(
                msgf_hbm.at[pl.ds(base * C, CHUNK * C)], data_v
            )
            pltpu.sync_copy(dst_hbm.at[pl.ds(base, CHUNK)], idx_v)

            def frame(k, carry3):
                dstvec = idx_v[pl.ds(k * 16, 16)]
                for jj in range(16):
                    d = dstvec[jj]
                    ld = d - lo

                    @pl.when((ld >= 0) & (ld < NHALF))
                    def _():
                        upd = data_v[pl.ds((k * 16 + jj) * C, 16)]
                        plsc.addupdate_scatter(acc_v, [ld * ST + iota], upd)
                        plsc.addupdate_scatter(
                            acc_v, [ld * ST + 16 + iota * 0], ones16,
                            mask=mask0,
                        )

                return carry3

            lax.fori_loop(0, CHUNK // 16, frame, 0)
            return carry2

        lax.fori_loop(0, NCHUNK, chunk, 0)
        pltpu.sync_copy(
            acc_v, parts_hbm.at[pl.ds(wid * NPASS * ACC_W + p * ACC_W, ACC_W)]
        )
        return carry

    lax.fori_loop(0, NPASS, one_pass, 0)


@functools.lru_cache(maxsize=None)
def _sc_calls():
    mesh = plsc.VectorSubcoreMesh(
        core_axis_name="c", subcore_axis_name="s",
        num_cores=NC, num_subcores=NS,
    )
    gather_call = pl.kernel(
        _gather_body,
        out_type=jax.ShapeDtypeStruct((E, C), jnp.float32),
        mesh=mesh,
        scratch_types=[
            pltpu.VMEM((SEG,), jnp.int32),
            pltpu.VMEM((SEG, 128), jnp.float32),
            pltpu.VMEM((CHUNK, C), jnp.float32),
            pltpu.SemaphoreType.DMA,
        ],
    )
    scatter_call = pl.kernel(
        _scatter_body,
        out_type=jax.ShapeDtypeStruct((NW * NPASS * ACC_W,), jnp.float32),
        mesh=mesh,
        scratch_types=[
            pltpu.VMEM((CHUNK,), jnp.int32),
            pltpu.VMEM((CHUNK * C,), jnp.float32),
            pltpu.VMEM((ACC_W,), jnp.float32),
        ],
        compiler_params=pltpu.CompilerParams(needs_layout_passes=False),
    )
    return gather_call, scatter_call


def _prelu(x, a):
    return jnp.where(x >= 0, x, a * x)


def _mlp_body(ea_ref, xj_ref, w1, b1, a1, w2, b2, a2, w3, b3, a3, w4, b4,
              r_ref, s_ref, msg_ref):
    f32 = jnp.float32
    h = _prelu(jnp.dot(ea_ref[...], w1[...], preferred_element_type=f32)
               + b1[...], a1[0, 0])
    h = _prelu(jnp.dot(h, w2[...], preferred_element_type=f32)
               + b2[...], a2[0, 0])
    h = _prelu(jnp.dot(h, w3[...], preferred_element_type=f32)
               + b3[...], a3[0, 0])
    w = jnp.dot(h, w4[...], preferred_element_type=f32) + b4[...]
    xrep = jnp.dot(xj_ref[...], r_ref[...], preferred_element_type=f32)
    msg_ref[...] = jnp.dot(w * xrep, s_ref[...], preferred_element_type=f32)


def _combine_body(parts_ref, xr_ref, root, bias, aout, out_ref):
    acc = jnp.sum(parts_ref[...], axis=0)
    cnt = jnp.maximum(acc[:, C:C + 1], 1.0)
    mean = acc[:, :C] / cnt
    out = mean + jnp.dot(xr_ref[...], root[...],
                         preferred_element_type=jnp.float32) + bias[...]
    out_ref[...] = _prelu(out, aout[0, 0])


BE = 3200
BN = 200


def _full(shape):
    return pl.BlockSpec(shape, lambda i: (0,) * len(shape))


def kernel(xr, edge_index, edge_attr, W1, b1, a1, W2, b2, a2, W3, b3, a3,
           W4, b4, root, bias, a_out):
    f32 = jnp.float32
    src = edge_index[0]
    dst = edge_index[1]

    gather_call, scatter_call = _sc_calls()
    xrp = jnp.pad(xr, ((0, 0), (0, 128 - C)))
    xj = gather_call(xrp, src)

    cc = jnp.arange(C * C, dtype=jnp.int32)
    R = (cc[None, :] // C == jnp.arange(C, dtype=jnp.int32)[:, None]).astype(f32)
    S = (cc[:, None] % C == jnp.arange(C, dtype=jnp.int32)[None, :]).astype(f32)

    edge_spec = pl.BlockSpec((BE, C), lambda i: (i, 0))
    msg = pl.pallas_call(
        _mlp_body,
        grid=(E // BE,),
        in_specs=[
            edge_spec, edge_spec,
            _full((DE, KW)), _full((1, KW)), _full((1, 1)),
            _full((KW, KW)), _full((1, KW)), _full((1, 1)),
            _full((KW, KW)), _full((1, KW)), _full((1, 1)),
            _full((KW, C * C)), _full((1, C * C)),
            _full((C, C * C)), _full((C * C, C)),
        ],
        out_specs=edge_spec,
        out_shape=jax.ShapeDtypeStruct((E, C), f32),
    )(edge_attr, xj,
      W1, b1.reshape(1, KW), a1.reshape(1, 1),
      W2, b2.reshape(1, KW), a2.reshape(1, 1),
      W3, b3.reshape(1, KW), a3.reshape(1, 1),
      W4, b4.reshape(1, C * C), R, S)

    parts = scatter_call(msg.reshape(E * C), dst)
    parts = parts.reshape(NW, N, ST)

    parts_spec = pl.BlockSpec((NW, BN, ST), lambda i: (0, i, 0))
    node_spec = pl.BlockSpec((BN, C), lambda i: (i, 0))
    out = pl.pallas_call(
        _combine_body,
        grid=(N // BN,),
        in_specs=[
            parts_spec, node_spec,
            _full((C, C)), _full((1, C)), _full((1, 1)),
        ],
        out_specs=node_spec,
        out_shape=jax.ShapeDtypeStruct((N, C), f32),
    )(parts, xr, root, bias.reshape(1, C), a_out.reshape(1, 1))
    return out


# speedup vs baseline: 2.6433x; 1.1088x over previous
import functools

import jax
import jax.numpy as jnp
from jax import lax
from jax.experimental import pallas as pl
from jax.experimental.pallas import tpu as pltpu
from jax.experimental.pallas import tpu_sc as plsc

N = 10000
E = 320000
C = 16
KW = 64
DE = 16

NC = 2
NS = 16
NW = NC * NS
EW = E // NW
SEG = 80
CHUNK = 400
NCHUNK = EW // CHUNK
SPC = CHUNK // SEG
ST = 17
NPASS = 2
NHALF = N // NPASS
ACC_W = NHALF * ST


def _gather_body(xrp_hbm, src_hbm, xj_hbm, idx_v, rowsp_v, out_v, sem):
    wid = lax.axis_index("s") * NC + lax.axis_index("c")

    def chunk(c, carry):
        base = wid * EW + c * CHUNK
        pltpu.sync_copy(src_hbm.at[pl.ds(base, CHUNK)], idx_v)
        descs = [
            pltpu.async_copy(
                xrp_hbm.at[idx_v.at[pl.ds(k * SEG, SEG)]],
                rowsp_v.at[pl.ds(k * SEG, SEG)],
                sem,
            )
            for k in range(SPC)
        ]
        for d in descs:
            d.wait()

        def extract(r, carry2):
            out_v[r] = rowsp_v[r, pl.ds(0, C)]
            return carry2

        lax.fori_loop(0, CHUNK, extract, 0)
        pltpu.sync_copy(out_v, xj_hbm.at[pl.ds(base, CHUNK)])
        return carry

    lax.fori_loop(0, NCHUNK, chunk, 0)


def _scatter_body(msgf_hbm, dst_hbm, parts_hbm, idx_v, data_v, acc_v):
    wid = lax.axis_index("s") * NC + lax.axis_index("c")
    iota = lax.iota(jnp.int32, 16)
    mask0 = iota == 0
    ones16 = jnp.ones((16,), jnp.float32)

    def one_pass(p, carry):
        lo = p * NHALF

        def zrow(i, carry2):
            acc_v[pl.ds(i * 16, 16)] = jnp.zeros((16,), jnp.float32)
            return carry2

        lax.fori_loop(0, ACC_W // 16, zrow, 0)

        def chunk(c, carry2):
            base = wid * EW + c * CHUNK
            pltpu.sync_copy(
                msgf_hbm.at[pl.ds(base * C, CHUNK * C)], data_v
            )
            pltpu.sync_copy(dst_hbm.at[pl.ds(base, CHUNK)], idx_v)

            def frame(k, carry3):
                dstvec = idx_v[pl.ds(k * 16, 16)]
                for jj in range(16):
                    d = dstvec[jj]
                    ld = d - lo

                    @pl.when((ld >= 0) & (ld < NHALF))
                    def _():
                        upd = data_v[pl.ds((k * 16 + jj) * C, 16)]
                        plsc.addupdate_scatter(acc_v, [ld * ST + iota], upd)
                        plsc.addupdate_scatter(
                            acc_v, [ld * ST + 16 + iota * 0], ones16,
                            mask=mask0,
                        )

                return carry3

            lax.fori_loop(0, CHUNK // 16, frame, 0)
            return carry2

        lax.fori_loop(0, NCHUNK, chunk, 0)
        pltpu.sync_copy(
            acc_v, parts_hbm.at[pl.ds(wid * NPASS * ACC_W + p * ACC_W, ACC_W)]
        )
        return carry

    lax.fori_loop(0, NPASS, one_pass, 0)


@functools.lru_cache(maxsize=None)
def _sc_calls():
    mesh = plsc.VectorSubcoreMesh(
        core_axis_name="c", subcore_axis_name="s",
        num_cores=NC, num_subcores=NS,
    )
    gather_call = pl.kernel(
        _gather_body,
        out_type=jax.ShapeDtypeStruct((E, C), jnp.float32),
        mesh=mesh,
        scratch_types=[
            pltpu.VMEM((CHUNK,), jnp.int32),
            pltpu.VMEM((CHUNK, 128), jnp.float32),
            pltpu.VMEM((CHUNK, C), jnp.float32),
            pltpu.SemaphoreType.DMA,
        ],
    )
    scatter_call = pl.kernel(
        _scatter_body,
        out_type=jax.ShapeDtypeStruct((NW * NPASS * ACC_W,), jnp.float32),
        mesh=mesh,
        scratch_types=[
            pltpu.VMEM((CHUNK,), jnp.int32),
            pltpu.VMEM((CHUNK * C,), jnp.float32),
            pltpu.VMEM((ACC_W,), jnp.float32),
        ],
        compiler_params=pltpu.CompilerParams(needs_layout_passes=False),
    )
    return gather_call, scatter_call


def _prelu(x, a):
    return jnp.where(x >= 0, x, a * x)


def _mlp_body(ea_ref, xj_ref, w1, b1, a1, w2, b2, a2, w3, b3, a3, w4, b4,
              r_ref, s_ref, msg_ref):
    f32 = jnp.float32
    h = _prelu(jnp.dot(ea_ref[...], w1[...], preferred_element_type=f32)
               + b1[...], a1[0, 0])
    h = _prelu(jnp.dot(h, w2[...], preferred_element_type=f32)
               + b2[...], a2[0, 0])
    h = _prelu(jnp.dot(h, w3[...], preferred_element_type=f32)
               + b3[...], a3[0, 0])
    w = jnp.dot(h, w4[...], preferred_element_type=f32) + b4[...]
    xrep = jnp.dot(xj_ref[...], r_ref[...], preferred_element_type=f32)
    msg_ref[...] = jnp.dot(w * xrep, s_ref[...], preferred_element_type=f32)


def _combine_body(parts_ref, xr_ref, root, bias, aout, out_ref):
    acc = jnp.sum(parts_ref[...], axis=0)
    cnt = jnp.maximum(acc[:, C:C + 1], 1.0)
    mean = acc[:, :C] / cnt
    out = mean + jnp.dot(xr_ref[...], root[...],
                         preferred_element_type=jnp.float32) + bias[...]
    out_ref[...] = _prelu(out, aout[0, 0])


BE = 3200
BN = 200


def _full(shape):
    return pl.BlockSpec(shape, lambda i: (0,) * len(shape))


def kernel(xr, edge_index, edge_attr, W1, b1, a1, W2, b2, a2, W3, b3, a3,
           W4, b4, root, bias, a_out):
    f32 = jnp.float32
    src = edge_index[0]
    dst = edge_index[1]

    gather_call, scatter_call = _sc_calls()
    xrp = jnp.pad(xr, ((0, 0), (0, 128 - C)))
    xj = gather_call(xrp, src)

    cc = jnp.arange(C * C, dtype=jnp.int32)
    R = (cc[None, :] // C == jnp.arange(C, dtype=jnp.int32)[:, None]).astype(f32)
    S = (cc[:, None] % C == jnp.arange(C, dtype=jnp.int32)[None, :]).astype(f32)

    edge_spec = pl.BlockSpec((BE, C), lambda i: (i, 0))
    msg = pl.pallas_call(
        _mlp_body,
        grid=(E // BE,),
        in_specs=[
            edge_spec, edge_spec,
            _full((DE, KW)), _full((1, KW)), _full((1, 1)),
            _full((KW, KW)), _full((1, KW)), _full((1, 1)),
            _full((KW, KW)), _full((1, KW)), _full((1, 1)),
            _full((KW, C * C)), _full((1, C * C)),
            _full((C, C * C)), _full((C * C, C)),
        ],
        out_specs=edge_spec,
        out_shape=jax.ShapeDtypeStruct((E, C), f32),
    )(edge_attr, xj,
      W1, b1.reshape(1, KW), a1.reshape(1, 1),
      W2, b2.reshape(1, KW), a2.reshape(1, 1),
      W3, b3.reshape(1, KW), a3.reshape(1, 1),
      W4, b4.reshape(1, C * C), R, S)

    parts = scatter_call(msg.reshape(E * C), dst)
    parts = parts.reshape(NW, N, ST)

    parts_spec = pl.BlockSpec((NW, BN, ST), lambda i: (0, i, 0))
    node_spec = pl.BlockSpec((BN, C), lambda i: (i, 0))
    out = pl.pallas_call(
        _combine_body,
        grid=(N // BN,),
        in_specs=[
            parts_spec, node_spec,
            _full((C, C)), _full((1, C)), _full((1, 1)),
        ],
        out_specs=node_spec,
        out_shape=jax.ShapeDtypeStruct((N, C), f32),
    )(parts, xr, root, bias.reshape(1, C), a_out.reshape(1, 1))
    return out


# speedup vs baseline: 2.6845x; 1.0156x over previous
import functools

import jax
import jax.numpy as jnp
from jax import lax
from jax.experimental import pallas as pl
from jax.experimental.pallas import tpu as pltpu
from jax.experimental.pallas import tpu_sc as plsc

N = 10000
E = 320000
C = 16
KW = 64
DE = 16

NC = 2
NS = 16
NW = NC * NS
EW = E // NW
SEG = 80
CHUNK = 400
NCHUNK = EW // CHUNK
SPC = CHUNK // SEG
ST = 17
NPASS = 2
NHALF = N // NPASS
ACC_W = NHALF * ST


def _gather_body(xrp_hbm, src_hbm, xj_hbm, idx_v, rowsp_v, out_v, sem):
    wid = lax.axis_index("s") * NC + lax.axis_index("c")

    def chunk(c, carry):
        base = wid * EW + c * CHUNK
        pltpu.sync_copy(src_hbm.at[pl.ds(base, CHUNK)], idx_v)
        descs = [
            pltpu.async_copy(
                xrp_hbm.at[idx_v.at[pl.ds(k * SEG, SEG)]],
                rowsp_v.at[pl.ds(k * SEG, SEG)],
                sem,
            )
            for k in range(SPC)
        ]
        for d in descs:
            d.wait()

        def extract(r, carry2):
            for u in range(8):
                out_v[r * 8 + u] = rowsp_v[r * 8 + u, pl.ds(0, C)]
            return carry2

        lax.fori_loop(0, CHUNK // 8, extract, 0)
        pltpu.sync_copy(out_v, xj_hbm.at[pl.ds(base, CHUNK)])
        return carry

    lax.fori_loop(0, NCHUNK, chunk, 0)


def _scatter_body(msgf_hbm, dst_hbm, parts_hbm, idx_v, data_v, acc_v):
    wid = lax.axis_index("s") * NC + lax.axis_index("c")
    iota = lax.iota(jnp.int32, 16)
    mask0 = iota == 0
    ones16 = jnp.ones((16,), jnp.float32)

    def one_pass(p, carry):
        lo = p * NHALF

        def zrow(i, carry2):
            acc_v[pl.ds(i * 16, 16)] = jnp.zeros((16,), jnp.float32)
            return carry2

        lax.fori_loop(0, ACC_W // 16, zrow, 0)

        def chunk(c, carry2):
            base = wid * EW + c * CHUNK
            pltpu.sync_copy(
                msgf_hbm.at[pl.ds(base * C, CHUNK * C)], data_v
            )
            pltpu.sync_copy(dst_hbm.at[pl.ds(base, CHUNK)], idx_v)

            def frame(k, carry3):
                dstvec = idx_v[pl.ds(k * 16, 16)]
                for jj in range(16):
                    d = dstvec[jj]
                    ld = d - lo

                    @pl.when((ld >= 0) & (ld < NHALF))
                    def _():
                        upd = data_v[pl.ds((k * 16 + jj) * C, 16)]
                        plsc.addupdate_scatter(acc_v, [ld * ST + iota], upd)
                        plsc.addupdate_scatter(
                            acc_v, [ld * ST + 16 + iota * 0], ones16,
                            mask=mask0,
                        )

                return carry3

            lax.fori_loop(0, CHUNK // 16, frame, 0)
            return carry2

        lax.fori_loop(0, NCHUNK, chunk, 0)
        pltpu.sync_copy(
            acc_v, parts_hbm.at[pl.ds(wid * NPASS * ACC_W + p * ACC_W, ACC_W)]
        )
        return carry

    lax.fori_loop(0, NPASS, one_pass, 0)


@functools.lru_cache(maxsize=None)
def _sc_calls():
    mesh = plsc.VectorSubcoreMesh(
        core_axis_name="c", subcore_axis_name="s",
        num_cores=NC, num_subcores=NS,
    )
    gather_call = pl.kernel(
        _gather_body,
        out_type=jax.ShapeDtypeStruct((E, C), jnp.float32),
        mesh=mesh,
        scratch_types=[
            pltpu.VMEM((CHUNK,), jnp.int32),
            pltpu.VMEM((CHUNK, 128), jnp.float32),
            pltpu.VMEM((CHUNK, C), jnp.float32),
            pltpu.SemaphoreType.DMA,
        ],
    )
    scatter_call = pl.kernel(
        _scatter_body,
        out_type=jax.ShapeDtypeStruct((NW * NPASS * ACC_W,), jnp.float32),
        mesh=mesh,
        scratch_types=[
            pltpu.VMEM((CHUNK,), jnp.int32),
            pltpu.VMEM((CHUNK * C,), jnp.float32),
            pltpu.VMEM((ACC_W,), jnp.float32),
        ],
        compiler_params=pltpu.CompilerParams(needs_layout_passes=False),
    )
    return gather_call, scatter_call


def _prelu(x, a):
    return jnp.where(x >= 0, x, a * x)


def _mlp_body(ea_ref, xj_ref, w1, b1, a1, w2, b2, a2, w3, b3, a3, w4, b4,
              r_ref, s_ref, msg_ref):
    f32 = jnp.float32
    h = _prelu(jnp.dot(ea_ref[...], w1[...], preferred_element_type=f32)
               + b1[...], a1[0, 0])
    h = _prelu(jnp.dot(h, w2[...], preferred_element_type=f32)
               + b2[...], a2[0, 0])
    h = _prelu(jnp.dot(h, w3[...], preferred_element_type=f32)
               + b3[...], a3[0, 0])
    w = jnp.dot(h, w4[...], preferred_element_type=f32) + b4[...]
    xrep = jnp.dot(xj_ref[...], r_ref[...], preferred_element_type=f32)
    msg_ref[...] = jnp.dot(w * xrep, s_ref[...], preferred_element_type=f32)


def _combine_body(parts_ref, xr_ref, root, bias, aout, out_ref):
    acc = jnp.sum(parts_ref[...], axis=0)
    cnt = jnp.maximum(acc[:, C:C + 1], 1.0)
    mean = acc[:, :C] / cnt
    out = mean + jnp.dot(xr_ref[...], root[...],
                         preferred_element_type=jnp.float32) + bias[...]
    out_ref[...] = _prelu(out, aout[0, 0])


BE = 3200
BN = 200


def _full(shape):
    return pl.BlockSpec(shape, lambda i: (0,) * len(shape))


def kernel(xr, edge_index, edge_attr, W1, b1, a1, W2, b2, a2, W3, b3, a3,
           W4, b4, root, bias, a_out):
    f32 = jnp.float32
    src = edge_index[0]
    dst = edge_index[1]

    gather_call, scatter_call = _sc_calls()
    xrp = jnp.pad(xr, ((0, 0), (0, 128 - C)))
    xj = gather_call(xrp, src)

    cc = jnp.arange(C * C, dtype=jnp.int32)
    R = (cc[None, :] // C == jnp.arange(C, dtype=jnp.int32)[:, None]).astype(f32)
    S = (cc[:, None] % C == jnp.arange(C, dtype=jnp.int32)[None, :]).astype(f32)

    edge_spec = pl.BlockSpec((BE, C), lambda i: (i, 0))
    msg = pl.pallas_call(
        _mlp_body,
        grid=(E // BE,),
        in_specs=[
            edge_spec, edge_spec,
            _full((DE, KW)), _full((1, KW)), _full((1, 1)),
            _full((KW, KW)), _full((1, KW)), _full((1, 1)),
            _full((KW, KW)), _full((1, KW)), _full((1, 1)),
            _full((KW, C * C)), _full((1, C * C)),
            _full((C, C * C)), _full((C * C, C)),
        ],
        out_specs=edge_spec,
        out_shape=jax.ShapeDtypeStruct((E, C), f32),
    )(edge_attr, xj,
      W1, b1.reshape(1, KW), a1.reshape(1, 1),
      W2, b2.reshape(1, KW), a2.reshape(1, 1),
      W3, b3.reshape(1, KW), a3.reshape(1, 1),
      W4, b4.reshape(1, C * C), R, S)

    parts = scatter_call(msg.reshape(E * C), dst)
    parts = parts.reshape(NW, N, ST)

    parts_spec = pl.BlockSpec((NW, BN, ST), lambda i: (0, i, 0))
    node_spec = pl.BlockSpec((BN, C), lambda i: (i, 0))
    out = pl.pallas_call(
        _combine_body,
        grid=(N // BN,),
        in_specs=[
            parts_spec, node_spec,
            _full((C, C)), _full((1, C)), _full((1, 1)),
        ],
        out_specs=node_spec,
        out_shape=jax.ShapeDtypeStruct((N, C), f32),
    )(parts, xr, root, bias.reshape(1, C), a_out.reshape(1, 1))
    return out


# speedup vs baseline: 2.7944x; 1.0410x over previous
import functools

import jax
import jax.numpy as jnp
from jax import lax
from jax.experimental import pallas as pl
from jax.experimental.pallas import tpu as pltpu
from jax.experimental.pallas import tpu_sc as plsc

N = 10000
E = 320000
C = 16
KW = 64
DE = 16

NC = 2
NS = 16
NW = NC * NS
EW = E // NW
SEG = 80
CHUNK = 400
NCHUNK = EW // CHUNK
SPC = CHUNK // SEG
ST = 17
NPASS = 2
NHALF = N // NPASS
ACC_W = NHALF * ST


def _gather_body(xrp_hbm, src_hbm, xj_hbm, idx_v, rowsp_v, out_v, sem):
    wid = lax.axis_index("s") * NC + lax.axis_index("c")

    def chunk(c, carry):
        base = wid * EW + c * CHUNK
        pltpu.sync_copy(src_hbm.at[pl.ds(base, CHUNK)], idx_v)
        descs = [
            pltpu.async_copy(
                xrp_hbm.at[idx_v.at[pl.ds(k * SEG, SEG)]],
                rowsp_v.at[pl.ds(k * SEG, SEG)],
                sem,
            )
            for k in range(SPC)
        ]
        for d in descs:
            d.wait()

        def extract(r, carry2):
            for u in range(8):
                out_v[r * 8 + u] = rowsp_v[r * 8 + u, pl.ds(0, C)]
            return carry2

        lax.fori_loop(0, CHUNK // 8, extract, 0)
        pltpu.sync_copy(out_v, xj_hbm.at[pl.ds(base, CHUNK)])
        return carry

    lax.fori_loop(0, NCHUNK, chunk, 0)


def _scatter_body(msgf_hbm, dst_hbm, parts_hbm, idx_v, data_v, acc_v):
    wid = lax.axis_index("s") * NC + lax.axis_index("c")
    iota = lax.iota(jnp.int32, 16)
    mask0 = iota == 0
    ones16 = jnp.ones((16,), jnp.float32)

    def one_pass(p, carry):
        lo = p * NHALF

        def zrow(i, carry2):
            for u in range(8):
                acc_v[pl.ds((i * 8 + u) * 16, 16)] = jnp.zeros(
                    (16,), jnp.float32
                )
            return carry2

        lax.fori_loop(0, ACC_W // 128, zrow, 0)
        for u in range((ACC_W % 128 + 15) // 16):
            acc_v[pl.ds(min(ACC_W - 16, ACC_W // 128 * 128 + u * 16), 16)] = (
                jnp.zeros((16,), jnp.float32)
            )

        def chunk(c, carry2):
            base = wid * EW + c * CHUNK
            pltpu.sync_copy(
                msgf_hbm.at[pl.ds(base * C, CHUNK * C)], data_v
            )
            pltpu.sync_copy(dst_hbm.at[pl.ds(base, CHUNK)], idx_v)

            def frame(k, carry3):
                dstvec = idx_v[pl.ds(k * 16, 16)]
                for jj in range(16):
                    d = dstvec[jj]
                    ld = d - lo

                    @pl.when((ld >= 0) & (ld < NHALF))
                    def _():
                        upd = data_v[pl.ds((k * 16 + jj) * C, 16)]
                        plsc.addupdate_scatter(acc_v, [ld * ST + iota], upd)
                        plsc.addupdate_scatter(
                            acc_v, [ld * ST + 16 + iota * 0], ones16,
                            mask=mask0,
                        )

                return carry3

            lax.fori_loop(0, CHUNK // 16, frame, 0)
            return carry2

        lax.fori_loop(0, NCHUNK, chunk, 0)
        pltpu.sync_copy(
            acc_v, parts_hbm.at[pl.ds(wid * NPASS * ACC_W + p * ACC_W, ACC_W)]
        )
        return carry

    lax.fori_loop(0, NPASS, one_pass, 0)


@functools.lru_cache(maxsize=None)
def _sc_calls():
    mesh = plsc.VectorSubcoreMesh(
        core_axis_name="c", subcore_axis_name="s",
        num_cores=NC, num_subcores=NS,
    )
    gather_call = pl.kernel(
        _gather_body,
        out_type=jax.ShapeDtypeStruct((E, C), jnp.float32),
        mesh=mesh,
        scratch_types=[
            pltpu.VMEM((CHUNK,), jnp.int32),
            pltpu.VMEM((CHUNK, 128), jnp.float32),
            pltpu.VMEM((CHUNK, C), jnp.float32),
            pltpu.SemaphoreType.DMA,
        ],
    )
    scatter_call = pl.kernel(
        _scatter_body,
        out_type=jax.ShapeDtypeStruct((NW * NPASS * ACC_W,), jnp.float32),
        mesh=mesh,
        scratch_types=[
            pltpu.VMEM((CHUNK,), jnp.int32),
            pltpu.VMEM((CHUNK * C,), jnp.float32),
            pltpu.VMEM((ACC_W,), jnp.float32),
        ],
        compiler_params=pltpu.CompilerParams(needs_layout_passes=False),
    )
    return gather_call, scatter_call


def _prelu(x, a):
    return jnp.where(x >= 0, x, a * x)


def _mlp_body(ea_ref, xj_ref, w1, b1, a1, w2, b2, a2, w3, b3, a3, w4, b4,
              r_ref, s_ref, msg_ref):
    f32 = jnp.float32
    h = _prelu(jnp.dot(ea_ref[...], w1[...], preferred_element_type=f32)
               + b1[...], a1[0, 0])
    h = _prelu(jnp.dot(h, w2[...], preferred_element_type=f32)
               + b2[...], a2[0, 0])
    h = _prelu(jnp.dot(h, w3[...], preferred_element_type=f32)
               + b3[...], a3[0, 0])
    w = jnp.dot(h, w4[...], preferred_element_type=f32) + b4[...]
    xrep = jnp.dot(xj_ref[...], r_ref[...], preferred_element_type=f32)
    msg_ref[...] = jnp.dot(w * xrep, s_ref[...], preferred_element_type=f32)


def _combine_body(parts_ref, xr_ref, root, bias, aout, out_ref):
    acc = jnp.sum(parts_ref[...], axis=0)
    cnt = jnp.maximum(acc[:, C:C + 1], 1.0)
    mean = acc[:, :C] / cnt
    out = mean + jnp.dot(xr_ref[...], root[...],
                         preferred_element_type=jnp.float32) + bias[...]
    out_ref[...] = _prelu(out, aout[0, 0])


BE = 3200
BN = 200


def _full(shape):
    return pl.BlockSpec(shape, lambda i: (0,) * len(shape))


def kernel(xr, edge_index, edge_attr, W1, b1, a1, W2, b2, a2, W3, b3, a3,
           W4, b4, root, bias, a_out):
    f32 = jnp.float32
    src = edge_index[0]
    dst = edge_index[1]

    gather_call, scatter_call = _sc_calls()
    xrp = jnp.pad(xr, ((0, 0), (0, 128 - C)))
    xj = gather_call(xrp, src)

    cc = jnp.arange(C * C, dtype=jnp.int32)
    R = (cc[None, :] // C == jnp.arange(C, dtype=jnp.int32)[:, None]).astype(f32)
    S = (cc[:, None] % C == jnp.arange(C, dtype=jnp.int32)[None, :]).astype(f32)

    edge_spec = pl.BlockSpec((BE, C), lambda i: (i, 0))
    msg = pl.pallas_call(
        _mlp_body,
        grid=(E // BE,),
        in_specs=[
            edge_spec, edge_spec,
            _full((DE, KW)), _full((1, KW)), _full((1, 1)),
            _full((KW, KW)), _full((1, KW)), _full((1, 1)),
            _full((KW, KW)), _full((1, KW)), _full((1, 1)),
            _full((KW, C * C)), _full((1, C * C)),
            _full((C, C * C)), _full((C * C, C)),
        ],
        out_specs=edge_spec,
        out_shape=jax.ShapeDtypeStruct((E, C), f32),
    )(edge_attr, xj,
      W1, b1.reshape(1, KW), a1.reshape(1, 1),
      W2, b2.reshape(1, KW), a2.reshape(1, 1),
      W3, b3.reshape(1, KW), a3.reshape(1, 1),
      W4, b4.reshape(1, C * C), R, S)

    parts = scatter_call(msg.reshape(E * C), dst)
    parts = parts.reshape(NW, N, ST)

    parts_spec = pl.BlockSpec((NW, BN, ST), lambda i: (0, i, 0))
    node_spec = pl.BlockSpec((BN, C), lambda i: (i, 0))
    out = pl.pallas_call(
        _combine_body,
        grid=(N // BN,),
        in_specs=[
            parts_spec, node_spec,
            _full((C, C)), _full((1, C)), _full((1, 1)),
        ],
        out_specs=node_spec,
        out_shape=jax.ShapeDtypeStruct((N, C), f32),
    )(parts, xr, root, bias.reshape(1, C), a_out.reshape(1, 1))
    return out
